# baseline (speedup 1.0000x reference)
import functools

import jax
import jax.numpy as jnp
from jax.experimental import pallas as pl
from jax.experimental.pallas import tpu as pltpu

_EPSILON = 1e-07
_LANES = 128
_SUBLANES = 8
_GROUP = _SUBLANES * _LANES


def _f1_body(yp_hbm, yt_hbm, o_ref, acc_ref, *, num_tiles, tile_groups, n):
    acc_ref[...] = jnp.zeros_like(acc_ref)

    def _accum(yp_ref, yt_ref):
        p = yp_ref[...]
        tf = yt_ref[...].astype(jnp.float32)
        acc_ref[0] += jnp.sum(p, axis=0)
        acc_ref[1] += jnp.sum(tf, axis=0)
        acc_ref[2] += jnp.sum(tf * p, axis=0)

    block = (tile_groups, _SUBLANES, _LANES)
    pipeline = pltpu.emit_pipeline(
        _accum,
        grid=(num_tiles,),
        in_specs=[
            pl.BlockSpec(block, lambda t: (t, 0, 0),
                         pipeline_mode=pl.Buffered(buffer_count=8)),
            pl.BlockSpec(block, lambda t: (t, 0, 0),
                         pipeline_mode=pl.Buffered(buffer_count=8)),
        ],
    )
    pipeline(yp_hbm, yt_hbm)

    s_p = jnp.sum(acc_ref[0])
    s_t = jnp.sum(acc_ref[1])
    s_tp = jnp.sum(acc_ref[2])

    eps = jnp.float32(_EPSILON)
    n_f = jnp.float32(n)
    tp1 = s_tp
    fp1 = s_p - s_tp
    fn1 = s_t - s_tp
    tp0 = n_f - s_t - s_p + s_tp
    pr0 = tp0 / (tp0 + fn1 + eps)
    re0 = tp0 / (tp0 + fp1 + eps)
    pr1 = tp1 / (tp1 + fp1 + eps)
    re1 = tp1 / (tp1 + fn1 + eps)
    f1_0 = 2.0 * pr0 * re0 / (pr0 + re0 + eps)
    f1_1 = 2.0 * pr1 * re1 / (pr1 + re1 + eps)
    f1_0 = jnp.clip(f1_0, eps, 1.0 - eps)
    f1_1 = jnp.clip(f1_1, eps, 1.0 - eps)
    o_ref[0, 0] = 0.5 * (f1_0 + f1_1)


def kernel(y_pred, y_true):
    n = y_pred.shape[0]
    if y_pred.dtype != jnp.float32:
        y_pred = y_pred.astype(jnp.float32)
    if y_true.dtype.itemsize > 4:
        y_true = y_true.astype(jnp.int32)

    groups = -(-n // _GROUP)
    tile_groups = min(128, groups)
    num_tiles = -(-groups // tile_groups)
    total_groups = num_tiles * tile_groups

    padded = total_groups * _GROUP
    if padded != n:
        y_pred = jnp.pad(y_pred, (0, padded - n))
        y_true = jnp.pad(y_true, (0, padded - n))
    yp = y_pred.reshape(total_groups, _SUBLANES, _LANES)
    yt = y_true.reshape(total_groups, _SUBLANES, _LANES)

    n_bytes = yp.size * yp.dtype.itemsize + yt.size * yt.dtype.itemsize
    out = pl.pallas_call(
        functools.partial(_f1_body, num_tiles=num_tiles,
                          tile_groups=tile_groups, n=n),
        out_shape=jax.ShapeDtypeStruct((1, 1), jnp.float32),
        in_specs=[
            pl.BlockSpec(memory_space=pl.ANY),
            pl.BlockSpec(memory_space=pl.ANY),
        ],
        out_specs=pl.BlockSpec(memory_space=pltpu.SMEM),
        scratch_shapes=[pltpu.VMEM((3, _SUBLANES, _LANES), jnp.float32)],
        cost_estimate=pl.CostEstimate(
            flops=4 * yp.size, transcendentals=0, bytes_accessed=n_bytes),
    )(yp, yt)
    return out.reshape(())


# speedup vs baseline: 1.0531x; 1.0531x over previous
import functools

import jax
import jax.numpy as jnp
from jax.experimental import pallas as pl
from jax.experimental.pallas import tpu as pltpu

_EPSILON = 1e-07
_LANES = 128
_SUBLANES = 8
_GROUP = _SUBLANES * _LANES


def _f1_body(yp_hbm, yt_hbm, o_ref, acc_ref, *, num_tiles, tile_groups, n):
    acc_ref[...] = jnp.zeros_like(acc_ref)

    def _accum(yp_ref, yt_ref):
        p = yp_ref[...]
        tf = yt_ref[...].astype(jnp.float32)
        acc_ref[0] += jnp.sum(p, axis=0)
        acc_ref[1] += jnp.sum(tf, axis=0)
        acc_ref[2] += jnp.sum(tf * p, axis=0)

    block = (tile_groups, _SUBLANES, _LANES)
    pipeline = pltpu.emit_pipeline(
        _accum,
        grid=(num_tiles,),
        in_specs=[
            pl.BlockSpec(block, lambda t: (t, 0, 0),
                         pipeline_mode=pl.Buffered(buffer_count=6)),
            pl.BlockSpec(block, lambda t: (t, 0, 0),
                         pipeline_mode=pl.Buffered(buffer_count=6)),
        ],
    )
    pipeline(yp_hbm, yt_hbm)

    s_p = jnp.sum(acc_ref[0])
    s_t = jnp.sum(acc_ref[1])
    s_tp = jnp.sum(acc_ref[2])

    eps = jnp.float32(_EPSILON)
    n_f = jnp.float32(n)
    tp1 = s_tp
    fp1 = s_p - s_tp
    fn1 = s_t - s_tp
    tp0 = n_f - s_t - s_p + s_tp
    pr0 = tp0 / (tp0 + fn1 + eps)
    re0 = tp0 / (tp0 + fp1 + eps)
    pr1 = tp1 / (tp1 + fp1 + eps)
    re1 = tp1 / (tp1 + fn1 + eps)
    f1_0 = 2.0 * pr0 * re0 / (pr0 + re0 + eps)
    f1_1 = 2.0 * pr1 * re1 / (pr1 + re1 + eps)
    f1_0 = jnp.clip(f1_0, eps, 1.0 - eps)
    f1_1 = jnp.clip(f1_1, eps, 1.0 - eps)
    o_ref[0, 0] = 0.5 * (f1_0 + f1_1)


def kernel(y_pred, y_true):
    n = y_pred.shape[0]
    if y_pred.dtype != jnp.float32:
        y_pred = y_pred.astype(jnp.float32)
    if y_true.dtype.itemsize > 4:
        y_true = y_true.astype(jnp.int32)

    groups = -(-n // _GROUP)
    tile_groups = min(256, groups)
    num_tiles = -(-groups // tile_groups)
    total_groups = num_tiles * tile_groups

    padded = total_groups * _GROUP
    if padded != n:
        y_pred = jnp.pad(y_pred, (0, padded - n))
        y_true = jnp.pad(y_true, (0, padded - n))
    yp = y_pred.reshape(total_groups, _SUBLANES, _LANES)
    yt = y_true.reshape(total_groups, _SUBLANES, _LANES)

    n_bytes = yp.size * yp.dtype.itemsize + yt.size * yt.dtype.itemsize
    out = pl.pallas_call(
        functools.partial(_f1_body, num_tiles=num_tiles,
                          tile_groups=tile_groups, n=n),
        out_shape=jax.ShapeDtypeStruct((1, 1), jnp.float32),
        in_specs=[
            pl.BlockSpec(memory_space=pl.ANY),
            pl.BlockSpec(memory_space=pl.ANY),
        ],
        out_specs=pl.BlockSpec(memory_space=pltpu.SMEM),
        scratch_shapes=[pltpu.VMEM((3, _SUBLANES, _LANES), jnp.float32)],
        cost_estimate=pl.CostEstimate(
            flops=4 * yp.size, transcendentals=0, bytes_accessed=n_bytes),
    )(yp, yt)
    return out.reshape(())
